# baseline (speedup 1.0000x reference)
import jax
import jax.numpy as jnp
from jax.experimental import pallas as pl

VOCAB = 1000
CTX = 50
BB = 32


def _onehot_block(ids_ref, out_ref):
    ids = ids_ref[...]
    d = VOCAB + CTX
    c_iota = jax.lax.broadcasted_iota(jnp.int32, (BB, CTX, d), 2)
    n_iota = jax.lax.broadcasted_iota(jnp.int32, (BB, CTX, d), 1)
    hit = (c_iota == ids[:, :, None]) | (c_iota == n_iota + VOCAB)
    out_ref[...] = hit.astype(jnp.float32)


def kernel(token_ids):
    b, n = token_ids.shape
    d = VOCAB + CTX
    grid = (b // BB,)
    return pl.pallas_call(
        _onehot_block,
        grid=grid,
        in_specs=[pl.BlockSpec((BB, n), lambda i: (i, 0))],
        out_specs=pl.BlockSpec((BB, n, d), lambda i: (i, 0, 0)),
        out_shape=jax.ShapeDtypeStruct((b, n, d), jnp.float32),
    )(token_ids)
